# baseline (speedup 1.0000x reference)
import functools

import jax
import jax.numpy as jnp
from jax import lax
from jax.experimental import pallas as pl
from jax.experimental.pallas import tpu as pltpu
from jax.experimental.pallas import tpu_sc as plsc

B, N, D = 8, 4096, 256
BLK = 512
NW = 32
ROWS_PER_W = (B * N) // NW
CHUNK = 128
NCHUNK = ROWS_PER_W // CHUNK
PHI_PER_W = N // NW


def _rank_body(phi_row_ref, phi_col_ref, fidx_ref, acc_ref):
    ib = pl.program_id(0)
    jb = pl.program_id(1)
    phi_i = phi_row_ref[...]
    phi_j = phi_col_ref[...]

    @pl.when(jb == 0)
    def _():
        acc_ref[...] = jnp.zeros((1, BLK), jnp.int32)

    @pl.when(jb < ib)
    def _():
        le = (phi_j <= phi_i).astype(jnp.int32)
        acc_ref[...] += jnp.sum(le, axis=0, keepdims=True)

    @pl.when(jb > ib)
    def _():
        lt = (phi_j < phi_i).astype(jnp.int32)
        acc_ref[...] += jnp.sum(lt, axis=0, keepdims=True)

    @pl.when(jb == ib)
    def _():
        tri = (lax.broadcasted_iota(jnp.int32, (BLK, BLK), 0)
               < lax.broadcasted_iota(jnp.int32, (BLK, BLK), 1))
        less = (phi_j < phi_i) | ((phi_j == phi_i) & tri)
        acc_ref[...] += jnp.sum(less.astype(jnp.int32), axis=0,
                                keepdims=True)

    @pl.when(jb == N // BLK - 1)
    def _():
        boff = lax.broadcasted_iota(jnp.int32, (B, BLK), 0) * N
        fidx_ref[...] = jnp.broadcast_to(acc_ref[...], (B, BLK)) + boff


def _compute_flat_idx(phi_row, phi_col):
    return pl.pallas_call(
        _rank_body,
        grid=(N // BLK, N // BLK),
        in_specs=[
            pl.BlockSpec((1, BLK), lambda i, j: (0, i)),
            pl.BlockSpec((BLK, 1), lambda i, j: (j, 0)),
        ],
        out_specs=pl.BlockSpec((B, BLK), lambda i, j: (0, i)),
        out_shape=jax.ShapeDtypeStruct((B, N), jnp.int32),
        scratch_shapes=[pltpu.VMEM((1, BLK), jnp.int32)],
    )(phi_row, phi_col)


@functools.cache
def _make_sc_scatter():
    mesh = plsc.VectorSubcoreMesh(core_axis_name="c", subcore_axis_name="s")

    @functools.partial(
        pl.kernel,
        mesh=mesh,
        out_type=[
            jax.ShapeDtypeStruct((B * N, D), jnp.float32),
            jax.ShapeDtypeStruct((N, 128), jnp.float32),
        ],
        scratch_types=[
            pltpu.VMEM((NCHUNK, CHUNK), jnp.int32),
            pltpu.VMEM((1, PHI_PER_W), jnp.int32),
            pltpu.VMEM((PHI_PER_W, 128), jnp.float32),
            pltpu.VMEM((CHUNK, D), jnp.float32),
            pltpu.VMEM((CHUNK, D), jnp.float32),
            pltpu.VMEM((CHUNK, D), jnp.float32),
            pltpu.SemaphoreType.DMA,
            pltpu.SemaphoreType.DMA,
            pltpu.SemaphoreType.DMA,
            pltpu.SemaphoreType.DMA,
            pltpu.SemaphoreType.DMA,
            pltpu.SemaphoreType.DMA,
            pltpu.SemaphoreType.DMA,
        ],
    )
    def _sc_scatter(embed_hbm, fidx_hbm, phit_hbm, out_embed, out_phit,
                    idx_v, idxp_v, phi_buf, buf0, buf1, buf2,
                    sl0, sl1, sl2, ss0, ss1, ss2, sem_phi):
        bufs = (buf0, buf1, buf2)
        sem_ld = (sl0, sl1, sl2)
        sem_st = (ss0, ss1, ss2)
        wid = lax.axis_index("s") * 2 + lax.axis_index("c")
        rowbase = wid * ROWS_PER_W
        pltpu.sync_copy(fidx_hbm.at[pl.ds(wid * NCHUNK, NCHUNK)], idx_v)
        pltpu.sync_copy(fidx_hbm.at[pl.ds(wid, 1)], idxp_v)
        pltpu.sync_copy(phit_hbm.at[pl.ds(wid * PHI_PER_W, PHI_PER_W)],
                        phi_buf)
        phi_scat = pltpu.async_copy(
            phi_buf, out_phit.at[idxp_v.at[0]], sem_phi)

        def load(c):
            return pltpu.async_copy(
                embed_hbm.at[pl.ds(rowbase + c * CHUNK, CHUNK)],
                bufs[c % 3], sem_ld[c % 3])

        loads = [None] * NCHUNK
        scats = [None] * NCHUNK
        loads[0] = load(0)
        loads[1] = load(1)
        for c in range(NCHUNK):
            loads[c].wait()
            scats[c] = pltpu.async_copy(
                bufs[c % 3], out_embed.at[idx_v.at[c]], sem_st[c % 3])
            if c + 2 < NCHUNK:
                if c >= 1:
                    scats[c - 1].wait()
                loads[c + 2] = load(c + 2)
        for c in range(NCHUNK - 3, NCHUNK):
            scats[c].wait()
        phi_scat.wait()

    return _sc_scatter


def kernel(key_embed, key_phi):
    phi_row = key_phi[0:1, :]
    phi_col = key_phi[0].reshape(N, 1)
    fidx = _compute_flat_idx(phi_row, phi_col)
    embed_flat = key_embed.reshape(B * N, D)
    fidx2d = fidx.reshape((B * N) // CHUNK, CHUNK)
    phit = jnp.pad(key_phi.T, ((0, 0), (0, 128 - B)))
    emb_sorted, phit_sorted = _make_sc_scatter()(embed_flat, fidx2d, phit)
    return emb_sorted.reshape(B, N, D), phit_sorted[:, :B].T


# speedup vs baseline: 1.3492x; 1.3492x over previous
import functools

import jax
import jax.numpy as jnp
from jax import lax
from jax.experimental import pallas as pl
from jax.experimental.pallas import tpu as pltpu
from jax.experimental.pallas import tpu_sc as plsc

B, N, D = 8, 4096, 256
BLK = 512
NW = 32
ROWS_PER_W = (B * N) // NW
CHUNK = 128
NCHUNK = ROWS_PER_W // CHUNK
PHI_PER_W = N // NW


def _rank_body(phi_row_ref, phi_col_ref, fidx_ref, acc_ref):
    ib = pl.program_id(0)
    phi_i = phi_row_ref[...]
    acc_ref[...] = jnp.zeros((1, BLK), jnp.int32)
    for jb in range(N // BLK):
        phi_j = phi_col_ref[jb * BLK:(jb + 1) * BLK, :]

        @pl.when(jb < ib)
        def _():
            le = (phi_j <= phi_i).astype(jnp.int32)
            acc_ref[...] += jnp.sum(le, axis=0, keepdims=True)

        @pl.when(jb > ib)
        def _():
            lt = (phi_j < phi_i).astype(jnp.int32)
            acc_ref[...] += jnp.sum(lt, axis=0, keepdims=True)

        @pl.when(jb == ib)
        def _():
            tri = (lax.broadcasted_iota(jnp.int32, (BLK, BLK), 0)
                   < lax.broadcasted_iota(jnp.int32, (BLK, BLK), 1))
            less = (phi_j < phi_i) | ((phi_j == phi_i) & tri)
            acc_ref[...] += jnp.sum(less.astype(jnp.int32), axis=0,
                                    keepdims=True)

    boff = lax.broadcasted_iota(jnp.int32, (B, BLK), 0) * N
    fidx_ref[...] = jnp.broadcast_to(acc_ref[...], (B, BLK)) + boff


def _compute_flat_idx(phi_row, phi_col):
    return pl.pallas_call(
        _rank_body,
        grid=(N // BLK,),
        in_specs=[
            pl.BlockSpec((1, BLK), lambda i: (0, i)),
            pl.BlockSpec((N, 1), lambda i: (0, 0)),
        ],
        out_specs=pl.BlockSpec((B, BLK), lambda i: (0, i)),
        out_shape=jax.ShapeDtypeStruct((B, N), jnp.int32),
        scratch_shapes=[pltpu.VMEM((1, BLK), jnp.int32)],
    )(phi_row, phi_col)


@functools.cache
def _make_sc_scatter():
    mesh = plsc.VectorSubcoreMesh(core_axis_name="c", subcore_axis_name="s")

    @functools.partial(
        pl.kernel,
        mesh=mesh,
        out_type=[
            jax.ShapeDtypeStruct((B * N, D), jnp.float32),
            jax.ShapeDtypeStruct((N, 128), jnp.float32),
        ],
        scratch_types=[
            pltpu.VMEM((NCHUNK, CHUNK), jnp.int32),
            pltpu.VMEM((1, PHI_PER_W), jnp.int32),
            pltpu.VMEM((PHI_PER_W, 128), jnp.float32),
            pltpu.VMEM((CHUNK, D), jnp.float32),
            pltpu.VMEM((CHUNK, D), jnp.float32),
            pltpu.VMEM((CHUNK, D), jnp.float32),
            pltpu.SemaphoreType.DMA,
            pltpu.SemaphoreType.DMA,
            pltpu.SemaphoreType.DMA,
            pltpu.SemaphoreType.DMA,
            pltpu.SemaphoreType.DMA,
            pltpu.SemaphoreType.DMA,
            pltpu.SemaphoreType.DMA,
        ],
    )
    def _sc_scatter(embed_hbm, fidx_hbm, phit_hbm, out_embed, out_phit,
                    idx_v, idxp_v, phi_buf, buf0, buf1, buf2,
                    sl0, sl1, sl2, ss0, ss1, ss2, sem_phi):
        bufs = (buf0, buf1, buf2)
        sem_ld = (sl0, sl1, sl2)
        sem_st = (ss0, ss1, ss2)
        wid = lax.axis_index("s") * 2 + lax.axis_index("c")
        rowbase = wid * ROWS_PER_W
        pltpu.sync_copy(fidx_hbm.at[pl.ds(wid * NCHUNK, NCHUNK)], idx_v)
        pltpu.sync_copy(fidx_hbm.at[pl.ds(wid, 1)], idxp_v)
        pltpu.sync_copy(phit_hbm.at[pl.ds(wid * PHI_PER_W, PHI_PER_W)],
                        phi_buf)
        phi_scat = pltpu.async_copy(
            phi_buf, out_phit.at[idxp_v.at[0]], sem_phi)

        def load(c):
            return pltpu.async_copy(
                embed_hbm.at[pl.ds(rowbase + c * CHUNK, CHUNK)],
                bufs[c % 3], sem_ld[c % 3])

        loads = [None] * NCHUNK
        scats = [None] * NCHUNK
        loads[0] = load(0)
        loads[1] = load(1)
        for c in range(NCHUNK):
            loads[c].wait()
            scats[c] = pltpu.async_copy(
                bufs[c % 3], out_embed.at[idx_v.at[c]], sem_st[c % 3])
            if c + 2 < NCHUNK:
                if c >= 1:
                    scats[c - 1].wait()
                loads[c + 2] = load(c + 2)
        for c in range(NCHUNK - 3, NCHUNK):
            scats[c].wait()
        phi_scat.wait()

    return _sc_scatter


def kernel(key_embed, key_phi):
    phi_row = key_phi[0:1, :]
    phi_col = key_phi[0].reshape(N, 1)
    fidx = _compute_flat_idx(phi_row, phi_col)
    embed_flat = key_embed.reshape(B * N, D)
    fidx2d = fidx.reshape((B * N) // CHUNK, CHUNK)
    phit = jnp.pad(key_phi.T, ((0, 0), (0, 128 - B)))
    emb_sorted, phit_sorted = _make_sc_scatter()(embed_flat, fidx2d, phit)
    return emb_sorted.reshape(B, N, D), phit_sorted[:, :B].T


# speedup vs baseline: 1.3851x; 1.0266x over previous
import functools

import jax
import jax.numpy as jnp
from jax import lax
from jax.experimental import pallas as pl
from jax.experimental.pallas import tpu as pltpu
from jax.experimental.pallas import tpu_sc as plsc

B, N, D = 8, 4096, 256
BLK = 512
NW = 32
ROWS_PER_W = (B * N) // NW
CHUNK = 128
NCHUNK = ROWS_PER_W // CHUNK
PHI_PER_W = N // NW


def _rank_body(phi_row_ref, phi_full_ref, phi_col_ref, fidx_ref, phit_ref,
               acc_ref):
    ib = pl.program_id(0)
    phi_i = phi_row_ref[...]
    for jb in range(N // BLK):
        phi_j = phi_col_ref[jb * BLK:(jb + 1) * BLK, :]

        @pl.when(jb < ib)
        def _():
            le = (phi_j <= phi_i).astype(jnp.int32)
            acc_ref[jb:jb + 1, :] = jnp.sum(le, axis=0, keepdims=True)

        @pl.when(jb > ib)
        def _():
            lt = (phi_j < phi_i).astype(jnp.int32)
            acc_ref[jb:jb + 1, :] = jnp.sum(lt, axis=0, keepdims=True)

        @pl.when(jb == ib)
        def _():
            tri = (lax.broadcasted_iota(jnp.int32, (BLK, BLK), 0)
                   < lax.broadcasted_iota(jnp.int32, (BLK, BLK), 1))
            less = (phi_j < phi_i) | ((phi_j == phi_i) & tri)
            acc_ref[jb:jb + 1, :] = jnp.sum(less.astype(jnp.int32), axis=0,
                                            keepdims=True)

    rank = jnp.sum(acc_ref[...], axis=0, keepdims=True)
    boff = lax.broadcasted_iota(jnp.int32, (B, BLK), 0) * N
    fidx_ref[...] = jnp.broadcast_to(rank, (B, BLK)) + boff
    pt = jnp.transpose(phi_full_ref[...])
    phit_ref[...] = jnp.pad(pt, ((0, 0), (0, 128 - B)))


def _compute_flat_idx(key_phi, phi_col):
    return pl.pallas_call(
        _rank_body,
        grid=(N // BLK,),
        in_specs=[
            pl.BlockSpec((1, BLK), lambda i: (0, i)),
            pl.BlockSpec((B, BLK), lambda i: (0, i)),
            pl.BlockSpec((N, 1), lambda i: (0, 0)),
        ],
        out_specs=[
            pl.BlockSpec((B, BLK), lambda i: (0, i)),
            pl.BlockSpec((BLK, 128), lambda i: (i, 0)),
        ],
        out_shape=[
            jax.ShapeDtypeStruct((B, N), jnp.int32),
            jax.ShapeDtypeStruct((N, 128), jnp.float32),
        ],
        scratch_shapes=[pltpu.VMEM((N // BLK, BLK), jnp.int32)],
    )(key_phi[0:1, :], key_phi, phi_col)


@functools.cache
def _make_sc_scatter():
    mesh = plsc.VectorSubcoreMesh(core_axis_name="c", subcore_axis_name="s")

    @functools.partial(
        pl.kernel,
        mesh=mesh,
        out_type=[
            jax.ShapeDtypeStruct((B * N, D), jnp.float32),
            jax.ShapeDtypeStruct((N, 128), jnp.float32),
        ],
        scratch_types=[
            pltpu.VMEM((NCHUNK, CHUNK), jnp.int32),
            pltpu.VMEM((1, PHI_PER_W), jnp.int32),
            pltpu.VMEM((PHI_PER_W, 128), jnp.float32),
            pltpu.VMEM((CHUNK, D), jnp.float32),
            pltpu.VMEM((CHUNK, D), jnp.float32),
            pltpu.VMEM((CHUNK, D), jnp.float32),
            pltpu.SemaphoreType.DMA,
            pltpu.SemaphoreType.DMA,
            pltpu.SemaphoreType.DMA,
            pltpu.SemaphoreType.DMA,
            pltpu.SemaphoreType.DMA,
            pltpu.SemaphoreType.DMA,
            pltpu.SemaphoreType.DMA,
        ],
    )
    def _sc_scatter(embed_hbm, fidx_hbm, phit_hbm, out_embed, out_phit,
                    idx_v, idxp_v, phi_buf, buf0, buf1, buf2,
                    sl0, sl1, sl2, ss0, ss1, ss2, sem_phi):
        bufs = (buf0, buf1, buf2)
        sem_ld = (sl0, sl1, sl2)
        sem_st = (ss0, ss1, ss2)
        wid = lax.axis_index("s") * 2 + lax.axis_index("c")
        rowbase = wid * ROWS_PER_W
        pltpu.sync_copy(fidx_hbm.at[pl.ds(wid * NCHUNK, NCHUNK)], idx_v)
        pltpu.sync_copy(fidx_hbm.at[pl.ds(wid, 1)], idxp_v)
        pltpu.sync_copy(phit_hbm.at[pl.ds(wid * PHI_PER_W, PHI_PER_W)],
                        phi_buf)
        phi_scat = pltpu.async_copy(
            phi_buf, out_phit.at[idxp_v.at[0]], sem_phi)

        def load(c):
            return pltpu.async_copy(
                embed_hbm.at[pl.ds(rowbase + c * CHUNK, CHUNK)],
                bufs[c % 3], sem_ld[c % 3])

        loads = [None] * NCHUNK
        scats = [None] * NCHUNK
        loads[0] = load(0)
        loads[1] = load(1)
        for c in range(NCHUNK):
            loads[c].wait()
            scats[c] = pltpu.async_copy(
                bufs[c % 3], out_embed.at[idx_v.at[c]], sem_st[c % 3])
            if c + 2 < NCHUNK:
                if c >= 1:
                    scats[c - 1].wait()
                loads[c + 2] = load(c + 2)
        for c in range(NCHUNK - 3, NCHUNK):
            scats[c].wait()
        phi_scat.wait()

    return _sc_scatter


def kernel(key_embed, key_phi):
    phi_col = key_phi[0].reshape(N, 1)
    fidx, phit = _compute_flat_idx(key_phi, phi_col)
    embed_flat = key_embed.reshape(B * N, D)
    fidx2d = fidx.reshape((B * N) // CHUNK, CHUNK)
    emb_sorted, phit_sorted = _make_sc_scatter()(embed_flat, fidx2d, phit)
    return emb_sorted.reshape(B, N, D), phit_sorted[:, :B].T
